# baseline (speedup 1.0000x reference)
import jax
import jax.numpy as jnp
from jax.experimental import pallas as pl

N = 50000
NG = 500
NCONV = 3
NGAUSS = 50
CUTOFF = 6.0


def _head_kernel(mol_ref, w1_ref, b1_ref, w2_ref, b2_ref, out_ref):
    mol = mol_ref[...]
    t = jax.nn.softplus(mol @ w1_ref[...] + b1_ref[...][None, :])
    out_ref[...] = t @ w2_ref[...] + b2_ref[...][None, :]


def kernel(atomic_numbers, edge_index, batch, distances, embedding, emb_W, emb_b, conv_lin_W, conv_lin_b, conv_bn_g, conv_bn_b, conv_ln_g, conv_ln_b, fc1_W, fc1_b, out_W, out_b):
    offsets = jnp.linspace(0.0, CUTOFF, NGAUSS)
    coeff = -0.5 / (offsets[1] - offsets[0]) ** 2
    edge_attr = jnp.exp(coeff * (distances[:, None] - offsets[None, :]) ** 2)
    x = embedding[atomic_numbers - 1]
    h = x @ emb_W + emb_b
    src = edge_index[0]
    dst = edge_index[1]
    for l in range(NCONV):
        x_j = h[src]
        x_i = h[dst]
        z = jnp.concatenate([x_i, x_j, edge_attr], axis=1) @ conv_lin_W[l] + conv_lin_b[l]
        mu = z.mean(axis=0)
        var = z.var(axis=0)
        z = (z - mu) / jnp.sqrt(var + 1e-5) * conv_bn_g[l] + conv_bn_b[l]
        z1, z2 = jnp.split(z, 2, axis=1)
        m = jax.nn.sigmoid(z1) * jax.nn.softplus(z2)
        agg = jax.ops.segment_sum(m, dst, num_segments=N)
        ln_mu = agg.mean(axis=-1, keepdims=True)
        ln_var = agg.var(axis=-1, keepdims=True)
        agg_n = (agg - ln_mu) / jnp.sqrt(ln_var + 1e-5) * conv_ln_g[l] + conv_ln_b[l]
        h = jax.nn.softplus(agg_n + h)
    counts = jax.ops.segment_sum(jnp.ones((N,), dtype=h.dtype), batch, num_segments=NG)
    sums = jax.ops.segment_sum(h, batch, num_segments=NG)
    mol = sums / jnp.maximum(counts, 1.0)[:, None]
    out = pl.pallas_call(
        _head_kernel,
        out_shape=jax.ShapeDtypeStruct((NG, 1), jnp.float32),
    )(mol, fc1_W, fc1_b, out_W, out_b)
    return out


# speedup vs baseline: 1.8504x; 1.8504x over previous
import functools

import jax
import jax.numpy as jnp
from jax import lax
from jax.experimental import pallas as pl
from jax.experimental.pallas import tpu as pltpu
from jax.experimental.pallas import tpu_sc as plsc

N = 50000
E = 800000
NG = 500
ATOM_DIM = 92
AFL = 64
NCONV = 3
FC = 128
NGAUSS = 50
CUTOFF = 6.0
EPS = 1e-5
PAD = 128

_NC = 2
_NS = 16
_NW = _NC * _NS
_EPW = E // _NW
_EPS_SC = E // _NS
_C = 128
_KG = 3
_GG = _C * _KG
_NGRP_GATHER = _EPW // _GG
_REM_GATHER = _EPW - _NGRP_GATHER * _GG
_NGRP_SCAT = _EPS_SC // _C
_REM_SCAT = _EPS_SC - _NGRP_SCAT * _C

@functools.cache
def _sc_mesh():
    return plsc.VectorSubcoreMesh(core_axis_name="c", subcore_axis_name="s",
                                  num_cores=_NC, num_subcores=_NS)


_f32 = jnp.float32
_i32 = jnp.int32


def _gather_body(h_hbm, src_hbm, dst_hbm, xi_hbm, xj_hbm,
                 idx_i, idx_j, rows_i, rows_j, idx_ri, idx_rj, sem):
    cid = lax.axis_index("c")
    sid = lax.axis_index("s")
    wid = sid * _NC + cid
    base = wid * _EPW

    def group(g, carry):
        gb = base + g * _GG
        for b in range(_KG):
            pltpu.sync_copy(dst_hbm.at[pl.ds(gb + b * _C, _C)], idx_i.at[b])
            pltpu.sync_copy(src_hbm.at[pl.ds(gb + b * _C, _C)], idx_j.at[b])
        cps = []
        for b in range(_KG):
            cps.append(pltpu.async_copy(
                h_hbm.at[idx_i.at[b]], rows_i.at[pl.ds(b * _C, _C)], sem))
            cps.append(pltpu.async_copy(
                h_hbm.at[idx_j.at[b]], rows_j.at[pl.ds(b * _C, _C)], sem))
        for cp in cps:
            cp.wait()
        pltpu.sync_copy(rows_i, xi_hbm.at[pl.ds(gb, _GG)])
        pltpu.sync_copy(rows_j, xj_hbm.at[pl.ds(gb, _GG)])
        return carry

    lax.fori_loop(0, _NGRP_GATHER, group, 0, unroll=False)

    rb = base + _NGRP_GATHER * _GG
    pltpu.sync_copy(dst_hbm.at[pl.ds(rb, _REM_GATHER)], idx_ri)
    pltpu.sync_copy(src_hbm.at[pl.ds(rb, _REM_GATHER)], idx_rj)
    cp1 = pltpu.async_copy(h_hbm.at[idx_ri], rows_i.at[pl.ds(0, _REM_GATHER)], sem)
    cp2 = pltpu.async_copy(h_hbm.at[idx_rj], rows_j.at[pl.ds(0, _REM_GATHER)], sem)
    cp1.wait()
    cp2.wait()
    pltpu.sync_copy(rows_i.at[pl.ds(0, _REM_GATHER)], xi_hbm.at[pl.ds(rb, _REM_GATHER)])
    pltpu.sync_copy(rows_j.at[pl.ds(0, _REM_GATHER)], xj_hbm.at[pl.ds(rb, _REM_GATHER)])


@functools.cache
def _gather_kernel():
    return pl.kernel(
        _gather_body,
        out_type=(jax.ShapeDtypeStruct((E, PAD), _f32),
                  jax.ShapeDtypeStruct((E, PAD), _f32)),
        mesh=_sc_mesh(),
        scratch_types=[
            pltpu.VMEM((_KG, _C), _i32),
            pltpu.VMEM((_KG, _C), _i32),
            pltpu.VMEM((_GG, PAD), _f32),
            pltpu.VMEM((_GG, PAD), _f32),
            pltpu.VMEM((_REM_GATHER,), _i32),
            pltpu.VMEM((_REM_GATHER,), _i32),
            pltpu.SemaphoreType.DMA,
        ],
    )


def _gather(h, src, dst):
    return _gather_kernel()(h, src, dst)


_NR4 = 12800
_RPS = _NR4 // _NS


def _scatter_body(m2lo_hbm, m2hi_hbm, idx4_hbm, zeros_hbm,
                  agglo_hbm, agghi_hbm, idx, rows, idx_r, aggsh, sem):
    cid = lax.axis_index("c")
    sid = lax.axis_index("s")

    pltpu.sync_copy(zeros_hbm, aggsh.at[pl.ds(sid * _RPS, _RPS)])
    plsc.subcore_barrier()

    base = sid * _EPS_SC

    def sweep(m_hbm):
        def group(g, carry):
            gb = base + g * _C
            pltpu.sync_copy(idx4_hbm.at[pl.ds(gb, _C)], idx.at[0])
            pltpu.sync_copy(m_hbm.at[pl.ds(gb, _C)], rows)
            pltpu.sync_copy(rows, aggsh.at[idx.at[0]], add=True)
            return carry

        lax.fori_loop(0, _NGRP_SCAT, group, 0, unroll=False)
        rb = base + _NGRP_SCAT * _C
        pltpu.sync_copy(idx4_hbm.at[pl.ds(rb, _REM_SCAT)],
                        idx_r.at[0, pl.ds(0, _REM_SCAT)])
        pltpu.sync_copy(m_hbm.at[pl.ds(rb, _REM_SCAT)],
                        rows.at[pl.ds(0, _REM_SCAT)])
        pltpu.sync_copy(rows.at[pl.ds(0, _REM_SCAT)],
                        aggsh.at[idx_r.at[0, pl.ds(0, _REM_SCAT)]], add=True)

    @pl.when(cid == 0)
    def _():
        sweep(m2lo_hbm)

    @pl.when(cid == 1)
    def _():
        sweep(m2hi_hbm)

    plsc.subcore_barrier()

    rs = pl.ds(sid * _RPS, _RPS)

    @pl.when(cid == 0)
    def _():
        pltpu.sync_copy(aggsh.at[rs], agglo_hbm.at[rs])

    @pl.when(cid == 1)
    def _():
        pltpu.sync_copy(aggsh.at[rs], agghi_hbm.at[rs])


@functools.cache
def _scatter_kernel():
    return pl.kernel(
        _scatter_body,
        out_type=(jax.ShapeDtypeStruct((_NR4, PAD), _f32),
                  jax.ShapeDtypeStruct((_NR4, PAD), _f32)),
        mesh=_sc_mesh(),
        scratch_types=[
            pltpu.VMEM((1, _C), _i32),
            pltpu.VMEM((_C, PAD), _f32),
            pltpu.VMEM((1, _REM_SCAT), _i32),
            pltpu.VMEM_SHARED((_NR4, PAD), _f32),
            pltpu.SemaphoreType.DMA,
        ],
    )


def _scatter(m2lo, m2hi, idx4, zeros_sc):
    return _scatter_kernel()(m2lo, m2hi, idx4, zeros_sc)


def _softplus(x):
    return jnp.maximum(x, 0.0) + jnp.log1p(jnp.exp(-jnp.abs(x)))


def _sigmoid(x):
    return 1.0 / (1.0 + jnp.exp(-x))


def _edge_z(xi, xj, dist, w_ref, b_ref):
    step = CUTOFF / (NGAUSS - 1)
    offs = lax.broadcasted_iota(_i32, (1, NGAUSS), 1).astype(_f32) * step
    coeff = -0.5 / step ** 2
    g = jnp.exp(coeff * (dist[:, None] - offs) ** 2)
    z = (jnp.dot(xi[:, 0:AFL], w_ref[0:AFL, :], preferred_element_type=_f32)
         + jnp.dot(xj[:, 0:AFL], w_ref[AFL:2 * AFL, :],
                   preferred_element_type=_f32)
         + jnp.dot(g, w_ref[2 * AFL:2 * AFL + NGAUSS, :],
                   preferred_element_type=_f32)
         + b_ref[...])
    return z


_BE = 3200
_NEB = E // _BE


def _stats_kernel(xi_ref, xj_ref, dist_ref, w_ref, b_ref, s1_ref, s2_ref):
    i = pl.program_id(0)

    @pl.when(i == 0)
    def _():
        s1_ref[...] = jnp.zeros_like(s1_ref)
        s2_ref[...] = jnp.zeros_like(s2_ref)

    dist = dist_ref[pl.ds(i * _BE, _BE)]
    z = _edge_z(xi_ref[...], xj_ref[...], dist, w_ref, b_ref)
    s1_ref[...] += jnp.sum(z, axis=0, keepdims=True)
    s2_ref[...] += jnp.sum(z * z, axis=0, keepdims=True)


def _tc_stats(xi, xj, dist, w, b):
    return pl.pallas_call(
        _stats_kernel,
        grid=(_NEB,),
        in_specs=[
            pl.BlockSpec((_BE, PAD), lambda i: (i, 0)),
            pl.BlockSpec((_BE, PAD), lambda i: (i, 0)),
            pl.BlockSpec((E,), lambda i: (0,)),
            pl.BlockSpec((2 * AFL + NGAUSS, FC), lambda i: (0, 0)),
            pl.BlockSpec((1, FC), lambda i: (0, 0)),
        ],
        out_specs=[
            pl.BlockSpec((1, FC), lambda i: (0, 0)),
            pl.BlockSpec((1, FC), lambda i: (0, 0)),
        ],
        out_shape=[
            jax.ShapeDtypeStruct((1, FC), _f32),
            jax.ShapeDtypeStruct((1, FC), _f32),
        ],
    )(xi, xj, dist, w, b)


def _edge_kernel(xi_ref, xj_ref, dist_ref, dst_ref, w_ref, b_ref, sc_ref,
                 sh_ref, mlo_ref, mhi_ref):
    i = pl.program_id(0)
    dist = dist_ref[pl.ds(i * _BE, _BE)]
    z = _edge_z(xi_ref[...], xj_ref[...], dist, w_ref, b_ref)
    zh = z * sc_ref[...] + sh_ref[...]
    m = _sigmoid(zh[:, 0:AFL]) * _softplus(zh[:, AFL:2 * AFL])
    k4 = (dst_ref[pl.ds(i * _BE, _BE)] % 4)[:, None]
    zero = jnp.zeros((_BE, AFL // 2), _f32)

    def place(mh):
        return jnp.concatenate(
            [jnp.where(k4 == k, mh, zero) for k in range(4)], axis=1)

    mlo_ref[...] = place(m[:, 0:AFL // 2])
    mhi_ref[...] = place(m[:, AFL // 2:AFL])


def _tc_edge(xi, xj, dist, dst, w, b, scale, shift):
    return pl.pallas_call(
        _edge_kernel,
        grid=(_NEB,),
        in_specs=[
            pl.BlockSpec((_BE, PAD), lambda i: (i, 0)),
            pl.BlockSpec((_BE, PAD), lambda i: (i, 0)),
            pl.BlockSpec((E,), lambda i: (0,)),
            pl.BlockSpec((E,), lambda i: (0,)),
            pl.BlockSpec((2 * AFL + NGAUSS, FC), lambda i: (0, 0)),
            pl.BlockSpec((1, FC), lambda i: (0, 0)),
            pl.BlockSpec((1, FC), lambda i: (0, 0)),
            pl.BlockSpec((1, FC), lambda i: (0, 0)),
        ],
        out_specs=[
            pl.BlockSpec((_BE, PAD), lambda i: (i, 0)),
            pl.BlockSpec((_BE, PAD), lambda i: (i, 0)),
        ],
        out_shape=[
            jax.ShapeDtypeStruct((E, PAD), _f32),
            jax.ShapeDtypeStruct((E, PAD), _f32),
        ],
    )(xi, xj, dist, dst, w, b, scale, shift)


def _didx_kernel(dst_ref, out_ref):
    out_ref[...] = dst_ref[...] // 4


def _tc_didx(dst):
    return pl.pallas_call(
        _didx_kernel,
        out_shape=jax.ShapeDtypeStruct((E,), _i32),
    )(dst)


NP = 51200
_BN = 2048
_NNB = NP // _BN


def _node_kernel(alo_ref, ahi_ref, h_ref, g_ref, b_ref, out_ref):
    agg = jnp.concatenate([alo_ref[...], ahi_ref[...]], axis=1)
    mu = jnp.mean(agg, axis=1, keepdims=True)
    var = jnp.mean(agg * agg, axis=1, keepdims=True) - mu * mu
    aggn = (agg - mu) * jax.lax.rsqrt(var + EPS) * g_ref[...] + b_ref[...]
    hn = _softplus(aggn + h_ref[:, 0:AFL])
    rid = (pl.program_id(0) * _BN
           + lax.broadcasted_iota(_i32, (_BN, 1), 0))
    hn = jnp.where(rid < N, hn, 0.0)
    out_ref[...] = jnp.concatenate([hn, jnp.zeros((_BN, PAD - AFL), _f32)],
                                   axis=1)


def _tc_node(alo, ahi, h, g, b):
    return pl.pallas_call(
        _node_kernel,
        grid=(_NNB,),
        in_specs=[
            pl.BlockSpec((_BN, AFL // 2), lambda i: (i, 0)),
            pl.BlockSpec((_BN, AFL // 2), lambda i: (i, 0)),
            pl.BlockSpec((_BN, PAD), lambda i: (i, 0)),
            pl.BlockSpec((1, AFL), lambda i: (0, 0)),
            pl.BlockSpec((1, AFL), lambda i: (0, 0)),
        ],
        out_specs=pl.BlockSpec((_BN, PAD), lambda i: (i, 0)),
        out_shape=jax.ShapeDtypeStruct((NP, PAD), _f32),
    )(alo, ahi, h, g, b)


def _init_kernel(an_ref, emb_ref, ew_ref, eb_ref, h_ref):
    an = an_ref[pl.ds(pl.program_id(0) * _BN, _BN)]
    onehot = (an[:, None] - 1
              == lax.broadcasted_iota(_i32, (_BN, 100), 1)).astype(_f32)
    emb2 = jnp.dot(emb_ref[...], ew_ref[...], preferred_element_type=_f32)
    hv = jnp.dot(onehot, emb2, preferred_element_type=_f32) + eb_ref[...]
    h_ref[...] = jnp.concatenate([hv, jnp.zeros((_BN, PAD - AFL), _f32)],
                                 axis=1)


def _tc_init(an, emb, ew, eb):
    return pl.pallas_call(
        _init_kernel,
        grid=(_NNB,),
        in_specs=[
            pl.BlockSpec((NP,), lambda i: (0,)),
            pl.BlockSpec((100, ATOM_DIM), lambda i: (0, 0)),
            pl.BlockSpec((ATOM_DIM, AFL), lambda i: (0, 0)),
            pl.BlockSpec((1, AFL), lambda i: (0, 0)),
        ],
        out_specs=pl.BlockSpec((_BN, PAD), lambda i: (i, 0)),
        out_shape=jax.ShapeDtypeStruct((NP, PAD), _f32),
    )(an, emb, ew, eb)


def _pool_kernel(batch_ref, h_ref, sums_ref, cnt_ref):
    i = pl.program_id(0)

    @pl.when(i == 0)
    def _():
        sums_ref[...] = jnp.zeros_like(sums_ref)
        cnt_ref[...] = jnp.zeros_like(cnt_ref)

    bt = batch_ref[pl.ds(i * _BN, _BN)]
    onehot = (bt[:, None]
              == lax.broadcasted_iota(_i32, (_BN, NG), 1)).astype(_f32)
    sums_ref[...] += lax.dot_general(onehot, h_ref[:, 0:AFL],
                                     (((0,), (0,)), ((), ())),
                                     preferred_element_type=_f32)
    cnt_ref[...] += lax.dot_general(onehot, jnp.ones((_BN, 1), _f32),
                                    (((0,), (0,)), ((), ())),
                                    preferred_element_type=_f32)


def _tc_pool(batch, h):
    return pl.pallas_call(
        _pool_kernel,
        grid=(_NNB,),
        in_specs=[
            pl.BlockSpec((NP,), lambda i: (0,)),
            pl.BlockSpec((_BN, PAD), lambda i: (i, 0)),
        ],
        out_specs=[
            pl.BlockSpec((NG, AFL), lambda i: (0, 0)),
            pl.BlockSpec((NG, 1), lambda i: (0, 0)),
        ],
        out_shape=[
            jax.ShapeDtypeStruct((NG, AFL), _f32),
            jax.ShapeDtypeStruct((NG, 1), _f32),
        ],
    )(batch, h)


def _head_kernel(sums_ref, cnt_ref, w1_ref, b1_ref, w2_ref, b2_ref, out_ref):
    cnt = jnp.maximum(cnt_ref[...], 1.0)
    mol = sums_ref[...] / cnt
    t = _softplus(jnp.dot(mol, w1_ref[...], preferred_element_type=_f32)
                  + b1_ref[...])
    out_ref[...] = (jnp.dot(t, w2_ref[...], preferred_element_type=_f32)
                    + b2_ref[...])


def _tc_head(sums, cnt, w1, b1, w2, b2):
    return pl.pallas_call(
        _head_kernel,
        out_shape=jax.ShapeDtypeStruct((NG, 1), _f32),
    )(sums, cnt, w1, b1, w2, b2)


def kernel(atomic_numbers, edge_index, batch, distances, embedding, emb_W,
           emb_b, conv_lin_W, conv_lin_b, conv_bn_g, conv_bn_b, conv_ln_g,
           conv_ln_b, fc1_W, fc1_b, out_W, out_b):
    src = edge_index[0].astype(_i32)
    dst = edge_index[1].astype(_i32)
    an = jnp.pad(atomic_numbers.astype(_i32), (0, NP - N))
    zeros_sc = jnp.zeros((_RPS, PAD), dtype=_f32)

    h = _tc_init(an, embedding, emb_W, emb_b.reshape(1, AFL))
    idx4 = _tc_didx(dst)

    for l in range(NCONV):
        w = conv_lin_W[l]
        b = conv_lin_b[l].reshape(1, FC)
        xi, xj = _gather(h, src, dst)
        s1, s2 = _tc_stats(xi, xj, distances, w, b)
        mu = s1 / E
        var = s2 / E - mu * mu
        rstd = lax.rsqrt(var + EPS)
        scale = conv_bn_g[l].reshape(1, FC) * rstd
        shift = conv_bn_b[l].reshape(1, FC) - mu * scale
        m2lo, m2hi = _tc_edge(xi, xj, distances, dst, w, b, scale, shift)
        plo, phi = _scatter(m2lo, m2hi, idx4, zeros_sc)
        alo = plo.reshape(NP, AFL // 2)
        ahi = phi.reshape(NP, AFL // 2)
        h = _tc_node(alo, ahi, h, conv_ln_g[l].reshape(1, AFL),
                     conv_ln_b[l].reshape(1, AFL))

    bt = jnp.pad(batch.astype(_i32), (0, NP - N), constant_values=NG)
    sums, cnt = _tc_pool(bt, h)
    out = _tc_head(sums, cnt, fc1_W, fc1_b.reshape(1, FC),
                   out_W, out_b.reshape(1, 1))
    return out
